# baseline (speedup 1.0000x reference)
import jax
import jax.numpy as jnp
from jax import lax
from jax.experimental import pallas as pl
from jax.experimental.pallas import tpu as pltpu
from jax.experimental.pallas import tpu_sc as plsc

D_MODEL = 64
MAX_LEN = 446
BATCH = 4096
HIST = 200
ROWS = BATCH * HIST
LANES = 16
BW = 512
TPAD = D_MODEL + 1
TE_WORDS = ((MAX_LEN * TPAD + 7) // 8) * 8


def _make_sc_call():
    mesh = plsc.VectorSubcoreMesh(core_axis_name="c", subcore_axis_name="s")
    nc, ns = mesh.num_cores, mesh.num_subcores
    nw = nc * ns
    nbs = BATCH // BW
    nhs = nw // nbs
    nch = HIST // nhs
    main = nch - (nch % 3)

    def body(x_hbm, ts_hbm, te_hbm, out_hbm,
             te_v, ts0, ts1, ts2, xb0, xb1, xb2,
             ste, si0, si1, si2, so0, so1, so2):
        wid = lax.axis_index("s") * nc + lax.axis_index("c")
        b0 = (wid % nbs) * BW
        h0 = (wid // nbs) * nch

        pltpu.async_copy(te_hbm, te_v, ste).wait()

        bufs = ((xb0, ts0, si0, so0), (xb1, ts1, si1, so1), (xb2, ts2, si2, so2))

        def start_in(c, b):
            xb, tsb, si, _ = bufs[b]
            h = h0 + c
            pltpu.async_copy(x_hbm.at[h, :, pl.ds(b0, BW)], xb, si)
            pltpu.async_copy(ts_hbm.at[pl.ds(h * BATCH + b0, BW)], tsb, si)

        def wait_in(b):
            xb, tsb, si, _ = bufs[b]
            pltpu.make_async_copy(x_hbm.at[0, :, pl.ds(0, BW)], xb, si).wait()
            pltpu.make_async_copy(ts_hbm.at[pl.ds(0, BW)], tsb, si).wait()

        def start_out(c, b):
            xb, _, _, so = bufs[b]
            h = h0 + c
            pltpu.async_copy(xb, out_hbm.at[h, :, pl.ds(b0, BW)], so)

        def wait_out(b):
            xb, _, _, so = bufs[b]
            pltpu.make_async_copy(x_hbm.at[0, :, pl.ds(0, BW)], xb, so).wait()

        def compute(b):
            xb, tsb, _, _ = bufs[b]
            rb = [tsb[pl.ds(g * LANES, LANES)] * TPAD for g in range(BW // LANES)]

            @pl.loop(0, D_MODEL)
            def _dloop(d):
                g = [plsc.load_gather(te_v, [rb[bg] + d])
                     for bg in range(BW // LANES)]
                for bg in range(BW // LANES):
                    plsc.addupdate(xb.at[d, pl.ds(bg * LANES, LANES)],
                                   g[bg])

        def step(c, b, first):
            wait_in(b)
            compute(b)
            start_out(c, b)
            nxt = c + 2

            def _pf():
                bp = (b + 2) % 3
                wait_out(bp)
                start_in(nxt, bp)

            if first:
                pl.when(jnp.logical_and(c >= 1, nxt < nch))(_pf)
            else:
                pl.when(nxt < nch)(_pf)

        start_in(0, 0)
        start_in(1, 1)
        start_in(2, 2)

        @pl.loop(0, main, step=3)
        def _chunks(c0):
            for b in range(3):
                step(c0 + b, b, b == 0)

        for c in range(main, nch):
            step(c, c % 3, False)

        wait_out((nch + 2) % 3)
        wait_out((nch + 1) % 3)
        wait_out(nch % 3)

    f32, i32 = jnp.float32, jnp.int32
    return pl.kernel(
        body,
        out_type=jax.ShapeDtypeStruct((HIST, D_MODEL, BATCH), f32),
        mesh=mesh,
        compiler_params=pltpu.CompilerParams(use_tc_tiling_on_sc=True,
                                             needs_layout_passes=False),
        scratch_types=[
            pltpu.VMEM((TE_WORDS,), f32),
            pltpu.VMEM((BW,), i32),
            pltpu.VMEM((BW,), i32),
            pltpu.VMEM((BW,), i32),
            pltpu.VMEM((D_MODEL, BW), f32),
            pltpu.VMEM((D_MODEL, BW), f32),
            pltpu.VMEM((D_MODEL, BW), f32),
            pltpu.SemaphoreType.DMA,
            pltpu.SemaphoreType.DMA,
            pltpu.SemaphoreType.DMA,
            pltpu.SemaphoreType.DMA,
            pltpu.SemaphoreType.DMA,
            pltpu.SemaphoreType.DMA,
            pltpu.SemaphoreType.DMA,
        ],
    )


def kernel(x, timestamp, te):
    xt = x.transpose(1, 2, 0)
    tst = timestamp.astype(jnp.int32).T.reshape(ROWS)
    tef = jnp.pad(te, ((0, 0), (0, TPAD - D_MODEL))).reshape(-1)
    tef = jnp.pad(tef, (0, TE_WORDS - MAX_LEN * TPAD))
    out_t = _make_sc_call()(xt, tst, tef)
    return out_t.transpose(2, 0, 1)


# speedup vs baseline: 1.0078x; 1.0078x over previous
import jax
import jax.numpy as jnp
from jax import lax
from jax.experimental import pallas as pl
from jax.experimental.pallas import tpu as pltpu
from jax.experimental.pallas import tpu_sc as plsc

D_MODEL = 64
MAX_LEN = 446
BATCH = 4096
HIST = 200
ROWS = BATCH * HIST
LANES = 16
BW = 256
TPAD = D_MODEL + 1
TE_WORDS = ((MAX_LEN * TPAD + 7) // 8) * 8


def _make_sc_call():
    mesh = plsc.VectorSubcoreMesh(core_axis_name="c", subcore_axis_name="s")
    nc, ns = mesh.num_cores, mesh.num_subcores
    nw = nc * ns
    nbs = BATCH // BW
    nhs = nw // nbs
    nch = HIST // nhs
    main = nch - (nch % 3)

    def body(x_hbm, ts_hbm, te_hbm, out_hbm,
             te_v, ts0, ts1, ts2, xb0, xb1, xb2,
             ste, si0, si1, si2, so0, so1, so2):
        wid = lax.axis_index("s") * nc + lax.axis_index("c")
        b0 = (wid % nbs) * BW
        h0 = (wid // nbs) * nch

        pltpu.async_copy(te_hbm, te_v, ste).wait()

        bufs = ((xb0, ts0, si0, so0), (xb1, ts1, si1, so1), (xb2, ts2, si2, so2))

        def start_in(c, b):
            xb, tsb, si, _ = bufs[b]
            h = h0 + c
            pltpu.async_copy(x_hbm.at[h, :, pl.ds(b0, BW)], xb, si)
            pltpu.async_copy(ts_hbm.at[pl.ds(h * BATCH + b0, BW)], tsb, si)

        def wait_in(b):
            xb, tsb, si, _ = bufs[b]
            pltpu.make_async_copy(x_hbm.at[0, :, pl.ds(0, BW)], xb, si).wait()
            pltpu.make_async_copy(ts_hbm.at[pl.ds(0, BW)], tsb, si).wait()

        def start_out(c, b):
            xb, _, _, so = bufs[b]
            h = h0 + c
            pltpu.async_copy(xb, out_hbm.at[h, :, pl.ds(b0, BW)], so)

        def wait_out(b):
            xb, _, _, so = bufs[b]
            pltpu.make_async_copy(x_hbm.at[0, :, pl.ds(0, BW)], xb, so).wait()

        def compute(b):
            xb, tsb, _, _ = bufs[b]
            rb = [tsb[pl.ds(g * LANES, LANES)] * TPAD for g in range(BW // LANES)]

            @pl.loop(0, D_MODEL, step=2)
            def _dloop(d0):
                for dd in range(2):
                    d = d0 + dd
                    g = [plsc.load_gather(te_v, [rb[bg] + d])
                         for bg in range(BW // LANES)]
                    for bg in range(BW // LANES):
                        plsc.addupdate(xb.at[d, pl.ds(bg * LANES, LANES)],
                                       g[bg])

        def step(c, b, first):
            wait_in(b)
            compute(b)
            start_out(c, b)
            nxt = c + 2

            def _pf():
                bp = (b + 2) % 3
                wait_out(bp)
                start_in(nxt, bp)

            if first:
                pl.when(jnp.logical_and(c >= 1, nxt < nch))(_pf)
            else:
                pl.when(nxt < nch)(_pf)

        start_in(0, 0)
        start_in(1, 1)
        start_in(2, 2)

        @pl.loop(0, main, step=3)
        def _chunks(c0):
            for b in range(3):
                step(c0 + b, b, b == 0)

        for c in range(main, nch):
            step(c, c % 3, False)

        wait_out((nch + 2) % 3)
        wait_out((nch + 1) % 3)
        wait_out(nch % 3)

    f32, i32 = jnp.float32, jnp.int32
    return pl.kernel(
        body,
        out_type=jax.ShapeDtypeStruct((HIST, D_MODEL, BATCH), f32),
        mesh=mesh,
        compiler_params=pltpu.CompilerParams(use_tc_tiling_on_sc=True,
                                             needs_layout_passes=False),
        scratch_types=[
            pltpu.VMEM((TE_WORDS,), f32),
            pltpu.VMEM((BW,), i32),
            pltpu.VMEM((BW,), i32),
            pltpu.VMEM((BW,), i32),
            pltpu.VMEM((D_MODEL, BW), f32),
            pltpu.VMEM((D_MODEL, BW), f32),
            pltpu.VMEM((D_MODEL, BW), f32),
            pltpu.SemaphoreType.DMA,
            pltpu.SemaphoreType.DMA,
            pltpu.SemaphoreType.DMA,
            pltpu.SemaphoreType.DMA,
            pltpu.SemaphoreType.DMA,
            pltpu.SemaphoreType.DMA,
            pltpu.SemaphoreType.DMA,
        ],
    )


def kernel(x, timestamp, te):
    xt = x.transpose(1, 2, 0)
    tst = timestamp.astype(jnp.int32).T.reshape(ROWS)
    tef = jnp.pad(te, ((0, 0), (0, TPAD - D_MODEL))).reshape(-1)
    tef = jnp.pad(tef, (0, TE_WORDS - MAX_LEN * TPAD))
    out_t = _make_sc_call()(xt, tst, tef)
    return out_t.transpose(2, 0, 1)
